# baseline (speedup 1.0000x reference)
import functools

import jax
import jax.numpy as jnp
import numpy as np
from jax import lax
from jax.experimental import pallas as pl
from jax.experimental.pallas import tpu as pltpu
from jax.experimental.pallas import tpu_sc as plsc

_H = 128
_NUM_LEVELS = 8
_F32 = jnp.float32
_BF16 = jnp.bfloat16
_BN_INV = np.float32(1.0 / np.sqrt(1.0 + 1e-5))


def _dotT(a, w):
    return lax.dot_general(a.astype(_BF16), w.astype(_BF16),
                           (((1,), (1,)), ((), ())),
                           preferred_element_type=_F32)


def _dotT32(a, w):
    return lax.dot_general(a, w, (((1,), (1,)), ((), ())),
                           preferred_element_type=_F32)


def _dot32(a, w):
    return jnp.dot(a, w, preferred_element_type=_F32)


def _sigmoid(x):
    return 0.5 * (jnp.tanh(0.5 * x) + 1.0)


def _gru_ones(msg, Wih, bih, Whh, bhh, ones_row):
    ghc = _dotT32(ones_row, Whh) + bhh
    gi = _dotT(msg, Wih) + bih
    r = _sigmoid(gi[:, :_H] + ghc[:, :_H])
    z = _sigmoid(gi[:, _H:2 * _H] + ghc[:, _H:2 * _H])
    n = jnp.tanh(gi[:, 2 * _H:] + r * ghc[:, 2 * _H:])
    return (1.0 - z) * n + z


def _main_body(*refs):
    (x_ref, fl_ref, g_ref) = refs[:3]
    pipes = [refs[3 + 10 * t: 13 + 10 * t] for t in range(4)]
    (Wp1, bp1, Wp2, bp2, Wp3, bp3, g1, be1, g2, be2) = refs[43:53]
    hs_ref, hf_ref, prob_ref = refs[53:56]

    xb = x_ref[...]
    fl = fl_ref[...]
    g = g_ref[...]
    act = (fl >= 1) & (fl <= _NUM_LEVELS - 1)
    m_and = act & (g == 1)
    m_not = act & (g == 2)

    ones_row = jnp.ones((1, _H), _F32)
    outs = []
    for t in range(4):
        W1, b1, W2, b2, W3, b3, Wih, bih, Whh, bhh = pipes[t]
        W1v = W1[...]
        if t >= 2:
            bias1 = b1[...] + _dotT32(ones_row, W1v[:, _H:])
            h = jnp.maximum(_dotT(xb, W1v[:, :_H]) + bias1, 0.0)
        else:
            h = jnp.maximum(_dotT(xb, W1v) + b1[...], 0.0)
        h = jnp.maximum(_dotT(h, W2[...]) + b2[...], 0.0)
        msg = _dotT(h, W3[...]) + b3[...]
        outs.append(_gru_ones(msg, Wih[...], bih[...], Whh[...], bhh[...],
                              ones_row))

    hs = jnp.where(m_and, outs[0], jnp.where(m_not, outs[1], 1.0))
    hf = jnp.where(m_and, outs[2], jnp.where(m_not, outs[3], 1.0))
    hs_ref[...] = hs
    hf_ref[...] = hf

    ph = _dotT32(hf, Wp1[...]) + bp1[...]
    ph = jnp.maximum(g1[...] * ph * _BN_INV + be1[...], 0.0)
    ph = _dotT32(ph, Wp2[...]) + bp2[...]
    ph = jnp.maximum(g2[...] * ph * _BN_INV + be2[...], 0.0)
    prob_ref[...] = _dot32(ph, Wp3[...]) + bp3[...]


def _full_spec(shape):
    nd = len(shape)
    return pl.BlockSpec(shape, lambda i, _nd=nd: (0,) * _nd)


def _row2(a):
    return a.reshape(1, -1)


def _main_call(x, fl2, g2, pipe_ws, prob_ws, block_n):
    n = x.shape[0]
    grid = (n // block_n,)
    weights = [w for pw in pipe_ws for w in pw] + list(prob_ws)
    in_specs = [
        pl.BlockSpec((block_n, _H), lambda i: (i, 0)),
        pl.BlockSpec((block_n, 1), lambda i: (i, 0)),
        pl.BlockSpec((block_n, 1), lambda i: (i, 0)),
    ] + [_full_spec(w.shape) for w in weights]
    out_specs = [
        pl.BlockSpec((block_n, _H), lambda i: (i, 0)),
        pl.BlockSpec((block_n, _H), lambda i: (i, 0)),
        pl.BlockSpec((block_n, 1), lambda i: (i, 0)),
    ]
    out_shape = [
        jax.ShapeDtypeStruct((n, _H), _F32),
        jax.ShapeDtypeStruct((n, _H), _F32),
        jax.ShapeDtypeStruct((n, 1), _F32),
    ]
    return pl.pallas_call(
        _main_body,
        grid=grid,
        in_specs=in_specs,
        out_specs=out_specs,
        out_shape=out_shape,
        compiler_params=pltpu.CompilerParams(
            dimension_semantics=("arbitrary",)),
    )(x, fl2, g2, *weights)


@functools.cache
def _make_sc_gather(num_rows, d):
    info = plsc.get_sparse_core_info()
    nw = info.num_cores * info.num_subcores
    b_per_w = num_rows // nw
    mesh = plsc.VectorSubcoreMesh(core_axis_name="c", subcore_axis_name="s")

    @functools.partial(
        pl.kernel,
        out_type=jax.ShapeDtypeStruct((num_rows, d), _F32),
        mesh=mesh,
        scratch_types=[
            pltpu.VMEM((b_per_w,), jnp.int32),
            pltpu.VMEM((b_per_w, d), _F32),
            pltpu.SemaphoreType.DMA,
        ],
    )
    def gather(table_hbm, idx_hbm, out_hbm, idx_v, rows_v, sem):
        wid = lax.axis_index("s") * info.num_cores + lax.axis_index("c")
        base = wid * b_per_w
        pltpu.sync_copy(idx_hbm.at[pl.ds(base, b_per_w)], idx_v)
        pltpu.async_copy(table_hbm.at[idx_v], rows_v, sem).wait()
        pltpu.sync_copy(rows_v, out_hbm.at[pl.ds(base, b_per_w)])

    return gather


def _rc_body(u_ref, v_ref, W1_ref, b1_ref, W2_ref, b2_ref, W3_ref, b3_ref,
             g1_ref, be1_ref, g2_ref, be2_ref, out_ref):
    W1 = W1_ref[...]
    h = _dotT32(u_ref[...], W1[:, :_H]) + _dotT32(v_ref[...], W1[:, _H:]) + b1_ref[...]
    h = jnp.maximum(g1_ref[...] * h * _BN_INV + be1_ref[...], 0.0)
    h = _dotT32(h, W2_ref[...]) + b2_ref[...]
    h = jnp.maximum(g2_ref[...] * h * _BN_INV + be2_ref[...], 0.0)
    out_ref[...] = _sigmoid(_dot32(h, W3_ref[...]) + b3_ref[...])


def _rc_call(u, v, rc_ws, block_p):
    p = u.shape[0]
    grid = (p // block_p,)
    in_specs = [
        pl.BlockSpec((block_p, _H), lambda i: (i, 0)),
        pl.BlockSpec((block_p, _H), lambda i: (i, 0)),
    ] + [_full_spec(w.shape) for w in rc_ws]
    return pl.pallas_call(
        _rc_body,
        grid=grid,
        in_specs=in_specs,
        out_specs=pl.BlockSpec((block_p, 1), lambda i: (i, 0)),
        out_shape=jax.ShapeDtypeStruct((p, 1), _F32),
        compiler_params=pltpu.CompilerParams(
            dimension_semantics=("arbitrary",)),
    )(u, v, *rc_ws)


def _pick_block(n, target):
    b = min(target, n)
    while n % b or b % 8:
        b -= 8 if b % 8 == 0 else b % 8
        if b <= 8:
            return 8
    return b


def kernel(x, forward_level, gate, rc_pair_index, params):
    n = x.shape[0]
    p = rc_pair_index.shape[1]
    fl2 = forward_level.astype(jnp.int32).reshape(n, 1)
    g2 = gate.astype(jnp.int32).reshape(n, 1)

    def pipe_w(mlp, gru):
        m, u = params[mlp], params[gru]
        return (m['W1'], _row2(m['b1']), m['W2'], _row2(m['b2']),
                m['W3'], _row2(m['b3']),
                u['W_ih'], _row2(u['b_ih']), u['W_hh'], _row2(u['b_hh']))

    pipe_ws = [
        pipe_w('aggr_and_strc', 'update_and_strc'),
        pipe_w('aggr_not_strc', 'update_not_strc'),
        pipe_w('aggr_and_func', 'update_and_func'),
        pipe_w('aggr_not_func', 'update_not_func'),
    ]
    rp = params['readout_prob']
    prob_ws = (rp['W1'], _row2(rp['b1']), rp['W2'], _row2(rp['b2']),
               rp['W3'].T, _row2(rp['b3']),
               _row2(rp['g1']), _row2(rp['be1']), _row2(rp['g2']), _row2(rp['be2']))

    block_n = _pick_block(n, 2000)
    hs, hf, prob = _main_call(x, fl2, g2, pipe_ws, prob_ws, block_n)

    info = plsc.get_sparse_core_info()
    align = 8 * info.num_cores * info.num_subcores
    idx = rc_pair_index.astype(jnp.int32).reshape(-1)
    pad = (-idx.shape[0]) % align
    if pad:
        idx = jnp.pad(idx, (0, pad))
    rows = _make_sc_gather(idx.shape[0], _H)(hs, idx)
    u = rows[:p]
    v = rows[p:2 * p]

    rr = params['readout_rc']
    rc_ws = (rr['W1'], _row2(rr['b1']), rr['W2'], _row2(rr['b2']),
             rr['W3'].T, _row2(rr['b3']),
             _row2(rr['g1']), _row2(rr['be1']), _row2(rr['g2']), _row2(rr['be2']))
    block_p = _pick_block(p, 2000)
    is_rc = _rc_call(u, v, rc_ws, block_p)
    return (hs, hf, prob, is_rc)


# speedup vs baseline: 1.0740x; 1.0740x over previous
import functools

import jax
import jax.numpy as jnp
import numpy as np
from jax import lax
from jax.experimental import pallas as pl
from jax.experimental.pallas import tpu as pltpu
from jax.experimental.pallas import tpu_sc as plsc

_H = 128
_NUM_LEVELS = 8
_F32 = jnp.float32
_BF16 = jnp.bfloat16
_BN_INV = np.float32(1.0 / np.sqrt(1.0 + 1e-5))


def _dot32(a, w):
    return jnp.dot(a, w, preferred_element_type=_F32)


def _sigmoid(x):
    return 0.5 * (jnp.tanh(0.5 * x) + 1.0)


def _fold_mlp(p, bn=False, func=False):
    W1, b1 = p['W1'], p['b1']
    if func:
        b1 = b1 + W1[:, _H:].sum(axis=1)
        W1 = W1[:, :_H]
    W2, b2, W3, b3 = p['W2'], p['b2'], p['W3'], p['b3']
    if bn:
        s1 = p['g1'] * _BN_INV
        b1 = s1 * b1 + p['be1']
        W1 = W1 * s1[:, None]
        s2 = p['g2'] * _BN_INV
        b2 = s2 * b2 + p['be2']
        W2 = W2 * s2[:, None]
    return W1.T, b1, W2.T, b2, W3.T, b3


def _fold_pipe(p_mlp, p_gru, func=False):
    W1, b1, W2, b2, W3, b3 = _fold_mlp(p_mlp, func=func)
    Wih, bih = p_gru['W_ih'], p_gru['b_ih']
    ghc = p_gru['W_hh'].sum(axis=1) + p_gru['b_hh']
    beta = bih + Wih @ b3
    beta = beta.at[:2 * _H].add(ghc[:2 * _H])
    cn = ghc[2 * _H:]
    return W1, b1, W2, b2, W3, Wih.T, beta, cn


def _fold_all(params):
    pipes = [
        _fold_pipe(params['aggr_and_strc'], params['update_and_strc']),
        _fold_pipe(params['aggr_not_strc'], params['update_not_strc']),
        _fold_pipe(params['aggr_and_func'], params['update_and_func'], func=True),
        _fold_pipe(params['aggr_not_func'], params['update_not_func'], func=True),
    ]
    W1cat = jnp.concatenate([p[0] for p in pipes], axis=1).astype(_BF16)
    b1 = jnp.stack([p[1] for p in pipes])[:, None, :]
    W2 = jnp.stack([p[2] for p in pipes]).astype(_BF16)
    b2 = jnp.stack([p[3] for p in pipes])[:, None, :]
    W3 = jnp.stack([p[4] for p in pipes]).astype(_BF16)
    Wih = jnp.stack([p[5] for p in pipes]).astype(_BF16)
    beta = jnp.stack([p[6] for p in pipes])[:, None, :]
    cn = jnp.stack([p[7] for p in pipes])[:, None, :]

    Wp1, bp1, Wp2, bp2, Wp3, bp3 = _fold_mlp(params['readout_prob'], bn=True)
    prob_w = (Wp1, bp1[None, :], Wp2, bp2[None, :], Wp3, bp3[None, :])

    Wr1, br1, Wr2, br2, Wr3, br3 = _fold_mlp(params['readout_rc'], bn=True)
    rc_w = (Wr1[:_H], Wr1[_H:], br1[None, :], Wr2, br2[None, :], Wr3, br3[None, :])
    return (W1cat, b1, W2, b2, W3, Wih, beta, cn), prob_w, rc_w


def _main_body(x_ref, fl_ref, g_ref,
               W1_ref, b1_ref, W2_ref, b2_ref, W3_ref, Wih_ref, beta_ref, cn_ref,
               Wp1_ref, bp1_ref, Wp2_ref, bp2_ref, Wp3_ref, bp3_ref,
               hs_ref, hf_ref, prob_ref):
    xb = x_ref[...].astype(_BF16)
    fl = fl_ref[...]
    g = g_ref[...]
    act = (fl >= 1) & (fl <= _NUM_LEVELS - 1)
    m_and = act & (g == 1)
    m_not = act & (g == 2)

    h1all = _dot32(xb, W1_ref[...])
    outs = []
    for t in range(4):
        h = jnp.maximum(h1all[:, t * _H:(t + 1) * _H] + b1_ref[t], 0.0)
        h = jnp.maximum(_dot32(h.astype(_BF16), W2_ref[t]) + b2_ref[t], 0.0)
        msg = _dot32(h.astype(_BF16), W3_ref[t])
        gi = _dot32(msg.astype(_BF16), Wih_ref[t]) + beta_ref[t]
        r = _sigmoid(gi[:, :_H])
        z = _sigmoid(gi[:, _H:2 * _H])
        n = jnp.tanh(gi[:, 2 * _H:] + r * cn_ref[t])
        outs.append((1.0 - z) * n + z)

    hs = jnp.where(m_and, outs[0], jnp.where(m_not, outs[1], 1.0))
    hf = jnp.where(m_and, outs[2], jnp.where(m_not, outs[3], 1.0))
    hs_ref[...] = hs
    hf_ref[...] = hf

    ph = jnp.maximum(_dot32(hf, Wp1_ref[...]) + bp1_ref[...], 0.0)
    ph = jnp.maximum(_dot32(ph, Wp2_ref[...]) + bp2_ref[...], 0.0)
    prob_ref[...] = _dot32(ph, Wp3_ref[...]) + bp3_ref[...]


def _full_spec(shape):
    nd = len(shape)
    return pl.BlockSpec(shape, lambda i, _nd=nd: (0,) * _nd)


def _main_call(x, fl2, g2, pipe_w, prob_w, block_n):
    n = x.shape[0]
    grid = (n // block_n,)
    weights = list(pipe_w) + list(prob_w)
    in_specs = [
        pl.BlockSpec((block_n, _H), lambda i: (i, 0)),
        pl.BlockSpec((block_n, 1), lambda i: (i, 0)),
        pl.BlockSpec((block_n, 1), lambda i: (i, 0)),
    ] + [_full_spec(w.shape) for w in weights]
    out_specs = [
        pl.BlockSpec((block_n, _H), lambda i: (i, 0)),
        pl.BlockSpec((block_n, _H), lambda i: (i, 0)),
        pl.BlockSpec((block_n, 1), lambda i: (i, 0)),
    ]
    out_shape = [
        jax.ShapeDtypeStruct((n, _H), _F32),
        jax.ShapeDtypeStruct((n, _H), _F32),
        jax.ShapeDtypeStruct((n, 1), _F32),
    ]
    return pl.pallas_call(
        _main_body,
        grid=grid,
        in_specs=in_specs,
        out_specs=out_specs,
        out_shape=out_shape,
        compiler_params=pltpu.CompilerParams(
            dimension_semantics=("arbitrary",)),
    )(x, fl2, g2, *weights)


@functools.cache
def _make_sc_gather(num_rows, d):
    info = plsc.get_sparse_core_info()
    nw = info.num_cores * info.num_subcores
    b_per_w = num_rows // nw
    mesh = plsc.VectorSubcoreMesh(core_axis_name="c", subcore_axis_name="s")

    @functools.partial(
        pl.kernel,
        out_type=jax.ShapeDtypeStruct((num_rows, d), _F32),
        mesh=mesh,
        scratch_types=[
            pltpu.VMEM((b_per_w,), jnp.int32),
            pltpu.VMEM((b_per_w, d), _F32),
            pltpu.SemaphoreType.DMA,
        ],
    )
    def gather(table_hbm, idx_hbm, out_hbm, idx_v, rows_v, sem):
        wid = lax.axis_index("s") * info.num_cores + lax.axis_index("c")
        base = wid * b_per_w
        pltpu.sync_copy(idx_hbm.at[pl.ds(base, b_per_w)], idx_v)
        pltpu.async_copy(table_hbm.at[idx_v], rows_v, sem).wait()
        pltpu.sync_copy(rows_v, out_hbm.at[pl.ds(base, b_per_w)])

    return gather


def _rc_body(u_ref, v_ref, A1_ref, B1_ref, b1_ref, W2_ref, b2_ref,
             W3_ref, b3_ref, out_ref):
    h = _dot32(u_ref[...], A1_ref[...]) + _dot32(v_ref[...], B1_ref[...]) + b1_ref[...]
    h = jnp.maximum(h, 0.0)
    h = jnp.maximum(_dot32(h, W2_ref[...]) + b2_ref[...], 0.0)
    out_ref[...] = _sigmoid(_dot32(h, W3_ref[...]) + b3_ref[...])


def _rc_call(rows, p, rc_w, block_p):
    grid = (p // block_p,)
    voff = p // block_p
    in_specs = [
        pl.BlockSpec((block_p, _H), lambda i: (i, 0)),
        pl.BlockSpec((block_p, _H), lambda i, _v=voff: (i + _v, 0)),
    ] + [_full_spec(w.shape) for w in rc_w]
    return pl.pallas_call(
        _rc_body,
        grid=grid,
        in_specs=in_specs,
        out_specs=pl.BlockSpec((block_p, 1), lambda i: (i, 0)),
        out_shape=jax.ShapeDtypeStruct((p, 1), _F32),
        compiler_params=pltpu.CompilerParams(
            dimension_semantics=("arbitrary",)),
    )(rows, rows, *rc_w)


def _pick_block(n, target):
    b = min(target, n)
    while n % b or b % 8:
        b -= 8 if b % 8 == 0 else b % 8
        if b <= 8:
            return 8
    return b


def kernel(x, forward_level, gate, rc_pair_index, params):
    n = x.shape[0]
    p = rc_pair_index.shape[1]
    pipe_w, prob_w, rc_w = _fold_all(params)
    fl2 = forward_level.astype(jnp.int32).reshape(n, 1)
    g2 = gate.astype(jnp.int32).reshape(n, 1)

    block_n = _pick_block(n, 2000)
    hs, hf, prob = _main_call(x, fl2, g2, pipe_w, prob_w, block_n)

    info = plsc.get_sparse_core_info()
    align = 8 * info.num_cores * info.num_subcores
    idx = rc_pair_index.astype(jnp.int32).reshape(-1)
    pad = (-idx.shape[0]) % align
    if pad:
        idx = jnp.pad(idx, (0, pad))
    rows = _make_sc_gather(idx.shape[0], _H)(hs, idx)

    block_p = _pick_block(p, 2000)
    is_rc = _rc_call(rows, p, rc_w, block_p)
    return (hs, hf, prob, is_rc)


# speedup vs baseline: 1.1700x; 1.0893x over previous
import functools

import jax
import jax.numpy as jnp
import numpy as np
from jax import lax
from jax.experimental import pallas as pl
from jax.experimental.pallas import tpu as pltpu
from jax.experimental.pallas import tpu_sc as plsc

_H = 128
_NUM_LEVELS = 8
_F32 = jnp.float32
_BF16 = jnp.bfloat16
_BN_INV = np.float32(1.0 / np.sqrt(1.0 + 1e-5))


def _dot32(a, w):
    return jnp.dot(a, w, preferred_element_type=_F32)


def _sigmoid(x):
    return 0.5 * (jnp.tanh(0.5 * x) + 1.0)


def _fold_mlp(p, bn=False, func=False):
    W1, b1 = p['W1'], p['b1']
    if func:
        b1 = b1 + W1[:, _H:].sum(axis=1)
        W1 = W1[:, :_H]
    W2, b2, W3, b3 = p['W2'], p['b2'], p['W3'], p['b3']
    if bn:
        s1 = p['g1'] * _BN_INV
        b1 = s1 * b1 + p['be1']
        W1 = W1 * s1[:, None]
        s2 = p['g2'] * _BN_INV
        b2 = s2 * b2 + p['be2']
        W2 = W2 * s2[:, None]
    return W1.T, b1, W2.T, b2, W3.T, b3


def _fold_pipe(p_mlp, p_gru, func=False):
    W1, b1, W2, b2, W3, b3 = _fold_mlp(p_mlp, func=func)
    Wih, bih = p_gru['W_ih'], p_gru['b_ih']
    ghc = p_gru['W_hh'].sum(axis=1) + p_gru['b_hh']
    beta = bih + Wih @ b3
    beta = beta.at[:2 * _H].add(ghc[:2 * _H])
    cn = ghc[2 * _H:]
    return W1, b1, W2, b2, W3, Wih.T, beta, cn


def _fold_all(params):
    pipes = [
        _fold_pipe(params['aggr_and_strc'], params['update_and_strc']),
        _fold_pipe(params['aggr_not_strc'], params['update_not_strc']),
        _fold_pipe(params['aggr_and_func'], params['update_and_func'], func=True),
        _fold_pipe(params['aggr_not_func'], params['update_not_func'], func=True),
    ]
    W1cat = jnp.concatenate([p[0] for p in pipes], axis=1)
    b1 = jnp.stack([p[1] for p in pipes])[:, None, :]
    W2 = jnp.stack([p[2] for p in pipes])
    b2 = jnp.stack([p[3] for p in pipes])[:, None, :]
    W3 = jnp.stack([p[4] for p in pipes])
    Wih = jnp.stack([p[5] for p in pipes])
    beta = jnp.stack([p[6] for p in pipes])[:, None, :]
    cn = jnp.stack([p[7] for p in pipes])[:, None, :]

    Wp1, bp1, Wp2, bp2, Wp3, bp3 = _fold_mlp(params['readout_prob'], bn=True)
    prob_w = (Wp1, bp1[None, :], Wp2, bp2[None, :], Wp3, bp3[None, :])

    Wr1, br1, Wr2, br2, Wr3, br3 = _fold_mlp(params['readout_rc'], bn=True)
    rc_w = (Wr1[:_H], Wr1[_H:], br1[None, :], Wr2, br2[None, :], Wr3, br3[None, :])
    return (W1cat, b1, W2, b2, W3, Wih, beta, cn), prob_w, rc_w


def _main_body(x_ref, fl_ref, g_ref,
               W1_ref, b1_ref, W2_ref, b2_ref, W3_ref, Wih_ref, beta_ref, cn_ref,
               Wp1_ref, bp1_ref, Wp2_ref, bp2_ref, Wp3_ref, bp3_ref,
               hs_ref, hf_ref, prob_ref):
    xb = x_ref[...]
    fl = fl_ref[...]
    g = g_ref[...]
    act = (fl >= 1) & (fl <= _NUM_LEVELS - 1)
    m_and = act & (g == 1)
    m_not = act & (g == 2)

    h1all = _dot32(xb, W1_ref[...])
    outs = []
    for t in range(4):
        h = jnp.maximum(h1all[:, t * _H:(t + 1) * _H] + b1_ref[t], 0.0)
        h = jnp.maximum(_dot32(h, W2_ref[t]) + b2_ref[t], 0.0)
        msg = _dot32(h, W3_ref[t])
        gi = _dot32(msg, Wih_ref[t]) + beta_ref[t]
        r = _sigmoid(gi[:, :_H])
        z = _sigmoid(gi[:, _H:2 * _H])
        n = jnp.tanh(gi[:, 2 * _H:] + r * cn_ref[t])
        outs.append((1.0 - z) * n + z)

    hs = jnp.where(m_and, outs[0], jnp.where(m_not, outs[1], 1.0))
    hf = jnp.where(m_and, outs[2], jnp.where(m_not, outs[3], 1.0))
    hs_ref[...] = hs
    hf_ref[...] = hf

    ph = jnp.maximum(_dot32(hf, Wp1_ref[...]) + bp1_ref[...], 0.0)
    ph = jnp.maximum(_dot32(ph, Wp2_ref[...]) + bp2_ref[...], 0.0)
    prob_ref[...] = _dot32(ph, Wp3_ref[...]) + bp3_ref[...]


def _full_spec(shape):
    nd = len(shape)
    return pl.BlockSpec(shape, lambda i, _nd=nd: (0,) * _nd)


def _main_call(x, fl2, g2, pipe_w, prob_w, block_n):
    n = x.shape[0]
    grid = (n // block_n,)
    weights = list(pipe_w) + list(prob_w)
    in_specs = [
        pl.BlockSpec((block_n, _H), lambda i: (i, 0)),
        pl.BlockSpec((block_n, 1), lambda i: (i, 0)),
        pl.BlockSpec((block_n, 1), lambda i: (i, 0)),
    ] + [_full_spec(w.shape) for w in weights]
    out_specs = [
        pl.BlockSpec((block_n, _H), lambda i: (i, 0)),
        pl.BlockSpec((block_n, _H), lambda i: (i, 0)),
        pl.BlockSpec((block_n, 1), lambda i: (i, 0)),
    ]
    out_shape = [
        jax.ShapeDtypeStruct((n, _H), _F32),
        jax.ShapeDtypeStruct((n, _H), _F32),
        jax.ShapeDtypeStruct((n, 1), _F32),
    ]
    return pl.pallas_call(
        _main_body,
        grid=grid,
        in_specs=in_specs,
        out_specs=out_specs,
        out_shape=out_shape,
        compiler_params=pltpu.CompilerParams(
            dimension_semantics=("arbitrary",)),
    )(x, fl2, g2, *weights)


@functools.cache
def _make_sc_gather(num_rows, d):
    info = plsc.get_sparse_core_info()
    nw = info.num_cores * info.num_subcores
    b_per_w = num_rows // nw
    mesh = plsc.VectorSubcoreMesh(core_axis_name="c", subcore_axis_name="s")

    @functools.partial(
        pl.kernel,
        out_type=jax.ShapeDtypeStruct((num_rows, d), _F32),
        mesh=mesh,
        scratch_types=[
            pltpu.VMEM((b_per_w,), jnp.int32),
            pltpu.VMEM((b_per_w, d), _F32),
            pltpu.SemaphoreType.DMA,
        ],
    )
    def gather(table_hbm, idx_hbm, out_hbm, idx_v, rows_v, sem):
        wid = lax.axis_index("s") * info.num_cores + lax.axis_index("c")
        base = wid * b_per_w
        pltpu.sync_copy(idx_hbm.at[pl.ds(base, b_per_w)], idx_v)
        pltpu.async_copy(table_hbm.at[idx_v], rows_v, sem).wait()
        pltpu.sync_copy(rows_v, out_hbm.at[pl.ds(base, b_per_w)])

    return gather


def _rc_body(u_ref, v_ref, A1_ref, B1_ref, b1_ref, W2_ref, b2_ref,
             W3_ref, b3_ref, out_ref):
    h = _dot32(u_ref[...], A1_ref[...]) + _dot32(v_ref[...], B1_ref[...]) + b1_ref[...]
    h = jnp.maximum(h, 0.0)
    h = jnp.maximum(_dot32(h, W2_ref[...]) + b2_ref[...], 0.0)
    out_ref[...] = _sigmoid(_dot32(h, W3_ref[...]) + b3_ref[...])


def _rc_call(rows, p, rc_w, block_p):
    grid = (p // block_p,)
    voff = p // block_p
    in_specs = [
        pl.BlockSpec((block_p, _H), lambda i: (i, 0)),
        pl.BlockSpec((block_p, _H), lambda i, _v=voff: (i + _v, 0)),
    ] + [_full_spec(w.shape) for w in rc_w]
    return pl.pallas_call(
        _rc_body,
        grid=grid,
        in_specs=in_specs,
        out_specs=pl.BlockSpec((block_p, 1), lambda i: (i, 0)),
        out_shape=jax.ShapeDtypeStruct((p, 1), _F32),
        compiler_params=pltpu.CompilerParams(
            dimension_semantics=("arbitrary",)),
    )(rows, rows, *rc_w)


def _pick_block(n, target):
    b = min(target, n)
    while n % b or b % 8:
        b -= 8 if b % 8 == 0 else b % 8
        if b <= 8:
            return 8
    return b


def kernel(x, forward_level, gate, rc_pair_index, params):
    n = x.shape[0]
    p = rc_pair_index.shape[1]
    pipe_w, prob_w, rc_w = _fold_all(params)
    fl2 = forward_level.astype(jnp.int32).reshape(n, 1)
    g2 = gate.astype(jnp.int32).reshape(n, 1)

    block_n = _pick_block(n, 2000)
    hs, hf, prob = _main_call(x, fl2, g2, pipe_w, prob_w, block_n)

    info = plsc.get_sparse_core_info()
    align = 8 * info.num_cores * info.num_subcores
    idx = rc_pair_index.astype(jnp.int32).reshape(-1)
    pad = (-idx.shape[0]) % align
    if pad:
        idx = jnp.pad(idx, (0, pad))
    rows = _make_sc_gather(idx.shape[0], _H)(hs, idx)

    block_p = _pick_block(p, 2000)
    is_rc = _rc_call(rows, p, rc_w, block_p)
    return (hs, hf, prob, is_rc)


# speedup vs baseline: 1.3277x; 1.1348x over previous
import functools

import jax
import jax.numpy as jnp
import numpy as np
from jax import lax
from jax.experimental import pallas as pl
from jax.experimental.pallas import tpu as pltpu
from jax.experimental.pallas import tpu_sc as plsc

_H = 128
_NUM_LEVELS = 8
_F32 = jnp.float32
_BF16 = jnp.bfloat16
_BN_INV = np.float32(1.0 / np.sqrt(1.0 + 1e-5))


def _dot32(a, w):
    return jnp.dot(a, w, preferred_element_type=_F32)


def _sigmoid(x):
    return 0.5 * (jnp.tanh(0.5 * x) + 1.0)


def _fold_mlp(p, bn=False, func=False):
    W1, b1 = p['W1'], p['b1']
    if func:
        b1 = b1 + W1[:, _H:].sum(axis=1)
        W1 = W1[:, :_H]
    W2, b2, W3, b3 = p['W2'], p['b2'], p['W3'], p['b3']
    if bn:
        s1 = p['g1'] * _BN_INV
        b1 = s1 * b1 + p['be1']
        W1 = W1 * s1[:, None]
        s2 = p['g2'] * _BN_INV
        b2 = s2 * b2 + p['be2']
        W2 = W2 * s2[:, None]
    return W1.T, b1, W2.T, b2, W3.T, b3


def _fold_pipe(p_mlp, p_gru, func=False):
    W1, b1, W2, b2, W3, b3 = _fold_mlp(p_mlp, func=func)
    Wih, bih = p_gru['W_ih'], p_gru['b_ih']
    ghc = p_gru['W_hh'].sum(axis=1) + p_gru['b_hh']
    beta = bih + Wih @ b3
    beta = beta.at[:2 * _H].add(ghc[:2 * _H])
    cn = ghc[2 * _H:]
    Wc = W3 @ Wih.T
    return W1, b1, W2, b2, Wc, beta, cn


def _fold_all(params):
    pipes = [
        _fold_pipe(params['aggr_and_strc'], params['update_and_strc']),
        _fold_pipe(params['aggr_not_strc'], params['update_not_strc']),
        _fold_pipe(params['aggr_and_func'], params['update_and_func'], func=True),
        _fold_pipe(params['aggr_not_func'], params['update_not_func'], func=True),
    ]
    W1cat = jnp.concatenate([p[0] for p in pipes], axis=1)
    b1 = jnp.stack([p[1] for p in pipes])[:, None, :]
    W2 = jnp.stack([p[2] for p in pipes])
    b2 = jnp.stack([p[3] for p in pipes])[:, None, :]
    Wc = jnp.stack([p[4] for p in pipes])
    beta = jnp.stack([p[5] for p in pipes])[:, None, :]
    cn = jnp.stack([p[6] for p in pipes])[:, None, :]

    Wp1, bp1, Wp2, bp2, Wp3, bp3 = _fold_mlp(params['readout_prob'], bn=True)
    prob_w = (Wp1, bp1[None, :], Wp2, bp2[None, :], Wp3, bp3[None, :])

    Wr1, br1, Wr2, br2, Wr3, br3 = _fold_mlp(params['readout_rc'], bn=True)
    rc_w = (Wr1[:_H], Wr1[_H:], br1[None, :], Wr2, br2[None, :], Wr3, br3[None, :])
    return (W1cat, b1, W2, b2, Wc, beta, cn), prob_w, rc_w


def _main_body(x_ref, sel_ref,
               W1_ref, b1_ref, W2_ref, b2_ref, Wc_ref, beta_ref, cn_ref,
               Wp1_ref, bp1_ref, Wp2_ref, bp2_ref, Wp3_ref, bp3_ref,
               hs_ref, hf_ref, prob_ref):
    xb = x_ref[...]
    sel = sel_ref[...].astype(jnp.int32)
    m_and = sel == 1
    m_not = sel == 2

    h1all = _dot32(xb, W1_ref[...])
    outs = []
    for t in range(4):
        h = jnp.maximum(h1all[:, t * _H:(t + 1) * _H] + b1_ref[t], 0.0)
        h = jnp.maximum(_dot32(h, W2_ref[t]) + b2_ref[t], 0.0)
        gi = _dot32(h, Wc_ref[t]) + beta_ref[t]
        r = _sigmoid(gi[:, :_H])
        z = _sigmoid(gi[:, _H:2 * _H])
        n = jnp.tanh(gi[:, 2 * _H:] + r * cn_ref[t])
        outs.append((1.0 - z) * n + z)

    hs = jnp.where(m_and, outs[0], jnp.where(m_not, outs[1], 1.0))
    hf = jnp.where(m_and, outs[2], jnp.where(m_not, outs[3], 1.0))
    hs_ref[...] = hs
    hf_ref[...] = hf

    ph = jnp.maximum(_dot32(hf, Wp1_ref[...]) + bp1_ref[...], 0.0)
    ph = jnp.maximum(_dot32(ph, Wp2_ref[...]) + bp2_ref[...], 0.0)
    prob_ref[...] = _dot32(ph, Wp3_ref[...]) + bp3_ref[...]


def _full_spec(shape):
    nd = len(shape)
    return pl.BlockSpec(shape, lambda i, _nd=nd: (0,) * _nd)


def _main_call(x, sel, pipe_w, prob_w, block_n):
    n = x.shape[0]
    grid = (n // block_n,)
    weights = list(pipe_w) + list(prob_w)
    in_specs = [
        pl.BlockSpec((block_n, _H), lambda i: (i, 0)),
        pl.BlockSpec((block_n, 1), lambda i: (i, 0)),
    ] + [_full_spec(w.shape) for w in weights]
    out_specs = [
        pl.BlockSpec((block_n, _H), lambda i: (i, 0)),
        pl.BlockSpec((block_n, _H), lambda i: (i, 0)),
        pl.BlockSpec((block_n, 1), lambda i: (i, 0)),
    ]
    out_shape = [
        jax.ShapeDtypeStruct((n, _H), _F32),
        jax.ShapeDtypeStruct((n, _H), _F32),
        jax.ShapeDtypeStruct((n, 1), _F32),
    ]
    return pl.pallas_call(
        _main_body,
        grid=grid,
        in_specs=in_specs,
        out_specs=out_specs,
        out_shape=out_shape,
        compiler_params=pltpu.CompilerParams(
            dimension_semantics=("arbitrary",)),
    )(x, sel, *weights)


@functools.cache
def _make_sc_gather(num_rows, d):
    info = plsc.get_sparse_core_info()
    nw = info.num_cores * info.num_subcores
    b_per_w = num_rows // nw
    mesh = plsc.VectorSubcoreMesh(core_axis_name="c", subcore_axis_name="s")

    @functools.partial(
        pl.kernel,
        out_type=jax.ShapeDtypeStruct((num_rows, d), _F32),
        mesh=mesh,
        scratch_types=[
            pltpu.VMEM((b_per_w,), jnp.int32),
            pltpu.VMEM((b_per_w, d), _F32),
            pltpu.SemaphoreType.DMA,
        ],
    )
    def gather(table_hbm, idx_hbm, out_hbm, idx_v, rows_v, sem):
        wid = lax.axis_index("s") * info.num_cores + lax.axis_index("c")
        base = wid * b_per_w
        pltpu.sync_copy(idx_hbm.at[pl.ds(base, b_per_w)], idx_v)
        pltpu.async_copy(table_hbm.at[idx_v], rows_v, sem).wait()
        pltpu.sync_copy(rows_v, out_hbm.at[pl.ds(base, b_per_w)])

    return gather


def _rc_body(u_ref, v_ref, A1_ref, B1_ref, b1_ref, W2_ref, b2_ref,
             W3_ref, b3_ref, out_ref):
    h = _dot32(u_ref[...], A1_ref[...]) + _dot32(v_ref[...], B1_ref[...]) + b1_ref[...]
    h = jnp.maximum(h, 0.0)
    h = jnp.maximum(_dot32(h, W2_ref[...]) + b2_ref[...], 0.0)
    out_ref[...] = _sigmoid(_dot32(h, W3_ref[...]) + b3_ref[...])


def _rc_call(rows, p, rc_w, block_p):
    grid = (p // block_p,)
    voff = p // block_p
    in_specs = [
        pl.BlockSpec((block_p, _H), lambda i: (i, 0)),
        pl.BlockSpec((block_p, _H), lambda i, _v=voff: (i + _v, 0)),
    ] + [_full_spec(w.shape) for w in rc_w]
    return pl.pallas_call(
        _rc_body,
        grid=grid,
        in_specs=in_specs,
        out_specs=pl.BlockSpec((block_p, 1), lambda i: (i, 0)),
        out_shape=jax.ShapeDtypeStruct((p, 1), _F32),
        compiler_params=pltpu.CompilerParams(
            dimension_semantics=("arbitrary",)),
    )(rows, rows, *rc_w)


def _pick_block(n, target, align=8):
    b = min(target, n)
    b -= b % align
    while b > align and (n % b or b % align):
        b -= align
    return max(b, align)


def kernel(x, forward_level, gate, rc_pair_index, params):
    n = x.shape[0]
    p = rc_pair_index.shape[1]
    pipe_w, prob_w, rc_w = _fold_all(params)
    fl = forward_level.astype(jnp.int32)
    g = gate.astype(jnp.int32)
    act = (fl >= 1) & (fl <= _NUM_LEVELS - 1)
    sel = jnp.where(act, g, 0).astype(jnp.int8).reshape(n, 1)

    block_n = _pick_block(n, 4000, align=32)
    hs, hf, prob = _main_call(x, sel, pipe_w, prob_w, block_n)

    info = plsc.get_sparse_core_info()
    align = 8 * info.num_cores * info.num_subcores
    idx = rc_pair_index.astype(jnp.int32).reshape(-1)
    pad = (-idx.shape[0]) % align
    if pad:
        idx = jnp.pad(idx, (0, pad))
    rows = _make_sc_gather(idx.shape[0], _H)(hs, idx)

    block_p = _pick_block(p, 2000)
    is_rc = _rc_call(rows, p, rc_w, block_p)
    return (hs, hf, prob, is_rc)


# speedup vs baseline: 1.3606x; 1.0248x over previous
import functools

import jax
import jax.numpy as jnp
import numpy as np
from jax import lax
from jax.experimental import pallas as pl
from jax.experimental.pallas import tpu as pltpu
from jax.experimental.pallas import tpu_sc as plsc

_H = 128
_NUM_LEVELS = 8
_F32 = jnp.float32
_BF16 = jnp.bfloat16
_BN_INV = np.float32(1.0 / np.sqrt(1.0 + 1e-5))


def _dot32(a, w):
    return jnp.dot(a, w, preferred_element_type=_F32)


def _sigmoid(x):
    return 0.5 * (jnp.tanh(0.5 * x) + 1.0)


def _fold_mlp(p, bn=False, func=False):
    W1, b1 = p['W1'], p['b1']
    if func:
        b1 = b1 + W1[:, _H:].sum(axis=1)
        W1 = W1[:, :_H]
    W2, b2, W3, b3 = p['W2'], p['b2'], p['W3'], p['b3']
    if bn:
        s1 = p['g1'] * _BN_INV
        b1 = s1 * b1 + p['be1']
        W1 = W1 * s1[:, None]
        s2 = p['g2'] * _BN_INV
        b2 = s2 * b2 + p['be2']
        W2 = W2 * s2[:, None]
    return W1.T, b1, W2.T, b2, W3.T, b3


def _fold_pipe(p_mlp, p_gru, func=False):
    W1, b1, W2, b2, W3, b3 = _fold_mlp(p_mlp, func=func)
    Wih, bih = p_gru['W_ih'], p_gru['b_ih']
    ghc = p_gru['W_hh'].sum(axis=1) + p_gru['b_hh']
    beta = bih + Wih @ b3
    beta = beta.at[:2 * _H].add(ghc[:2 * _H])
    cn = ghc[2 * _H:]
    Wc = W3 @ Wih.T
    return W1, b1, W2, b2, Wc, beta, cn


def _fold_all(params):
    pipes = [
        _fold_pipe(params['aggr_and_strc'], params['update_and_strc']),
        _fold_pipe(params['aggr_not_strc'], params['update_not_strc']),
        _fold_pipe(params['aggr_and_func'], params['update_and_func'], func=True),
        _fold_pipe(params['aggr_not_func'], params['update_not_func'], func=True),
    ]
    W1cat = jnp.concatenate([p[0] for p in pipes], axis=1)
    b1 = jnp.stack([p[1] for p in pipes])[:, None, :]
    W2 = jnp.stack([p[2] for p in pipes])
    b2 = jnp.stack([p[3] for p in pipes])[:, None, :]
    Wc = jnp.stack([p[4] for p in pipes])
    beta = jnp.stack([p[5] for p in pipes])[:, None, :]
    cn = jnp.stack([p[6] for p in pipes])[:, None, :]

    Wp1, bp1, Wp2, bp2, Wp3, bp3 = _fold_mlp(params['readout_prob'], bn=True)
    prob_w = (Wp1, bp1[None, :], Wp2, bp2[None, :], Wp3, bp3[None, :])

    Wr1, br1, Wr2, br2, Wr3, br3 = _fold_mlp(params['readout_rc'], bn=True)
    rc_w = (Wr1[:_H], Wr1[_H:], br1[None, :], Wr2, br2[None, :], Wr3, br3[None, :])
    return (W1cat, b1, W2, b2, Wc, beta, cn), prob_w, rc_w


def _main_body(x_ref, sel_ref,
               W1_ref, b1_ref, W2_ref, b2_ref, Wc_ref, beta_ref, cn_ref,
               Wp1_ref, bp1_ref, Wp2_ref, bp2_ref, Wp3_ref, bp3_ref,
               hs_ref, hf_ref, prob_ref, hs2_ref):
    xb = x_ref[...]
    sel = sel_ref[...].astype(jnp.int32)
    m_and = sel == 1
    m_not = sel == 2

    h1all = _dot32(xb, W1_ref[...])
    outs = []
    for t in range(4):
        h = jnp.maximum(h1all[:, t * _H:(t + 1) * _H] + b1_ref[t], 0.0)
        h = jnp.maximum(_dot32(h, W2_ref[t]) + b2_ref[t], 0.0)
        gi = _dot32(h, Wc_ref[t]) + beta_ref[t]
        r = _sigmoid(gi[:, :_H])
        z = _sigmoid(gi[:, _H:2 * _H])
        n = jnp.tanh(gi[:, 2 * _H:] + r * cn_ref[t])
        outs.append((1.0 - z) * n + z)

    hs = jnp.where(m_and, outs[0], jnp.where(m_not, outs[1], 1.0))
    hf = jnp.where(m_and, outs[2], jnp.where(m_not, outs[3], 1.0))
    hs_ref[...] = hs
    hf_ref[...] = hf
    hs2_ref[...] = hs

    ph = jnp.maximum(_dot32(hf, Wp1_ref[...]) + bp1_ref[...], 0.0)
    ph = jnp.maximum(_dot32(ph, Wp2_ref[...]) + bp2_ref[...], 0.0)
    prob_ref[...] = _dot32(ph, Wp3_ref[...]) + bp3_ref[...]


def _full_spec(shape):
    nd = len(shape)
    return pl.BlockSpec(shape, lambda i, _nd=nd: (0,) * _nd)


def _main_call(x, sel, pipe_w, prob_w, block_n):
    n = x.shape[0]
    grid = (n // block_n,)
    weights = list(pipe_w) + list(prob_w)
    in_specs = [
        pl.BlockSpec((block_n, _H), lambda i: (i, 0)),
        pl.BlockSpec((block_n, _H), lambda i: (i, 0)),
    ] + [_full_spec(w.shape) for w in weights]
    out_specs = [
        pl.BlockSpec((block_n, _H), lambda i: (i, 0)),
        pl.BlockSpec((block_n, _H), lambda i: (i, 0)),
        pl.BlockSpec((block_n, 1), lambda i: (i, 0)),
        pl.BlockSpec((block_n, _H), lambda i: (i, 0)),
    ]
    out_shape = [
        jax.ShapeDtypeStruct((n, _H), _F32),
        jax.ShapeDtypeStruct((n, _H), _F32),
        jax.ShapeDtypeStruct((n, 1), _F32),
        jax.ShapeDtypeStruct((n, _H), _F32),
    ]
    return pl.pallas_call(
        _main_body,
        grid=grid,
        in_specs=in_specs,
        out_specs=out_specs,
        out_shape=out_shape,
        compiler_params=pltpu.CompilerParams(
            dimension_semantics=("arbitrary",)),
    )(x, sel, *weights)


@functools.cache
def _make_sc_gather(num_rows, d):
    info = plsc.get_sparse_core_info()
    nw = info.num_cores * info.num_subcores
    b_per_w = num_rows // nw
    mesh = plsc.VectorSubcoreMesh(core_axis_name="c", subcore_axis_name="s")

    @functools.partial(
        pl.kernel,
        out_type=jax.ShapeDtypeStruct((num_rows, d), _F32),
        mesh=mesh,
        scratch_types=[
            pltpu.VMEM((b_per_w,), jnp.int32),
            pltpu.VMEM((b_per_w, d), _F32),
            pltpu.SemaphoreType.DMA,
        ],
    )
    def gather(table_hbm, idx_hbm, out_hbm, idx_v, rows_v, sem):
        wid = lax.axis_index("s") * info.num_cores + lax.axis_index("c")
        base = wid * b_per_w
        pltpu.sync_copy(idx_hbm.at[pl.ds(base, b_per_w)], idx_v)
        pltpu.async_copy(table_hbm.at[idx_v], rows_v, sem).wait()
        pltpu.sync_copy(rows_v, out_hbm.at[pl.ds(base, b_per_w)])

    return gather


def _rc_body(u_ref, v_ref, A1_ref, B1_ref, b1_ref, W2_ref, b2_ref,
             W3_ref, b3_ref, out_ref):
    h = _dot32(u_ref[...], A1_ref[...]) + _dot32(v_ref[...], B1_ref[...]) + b1_ref[...]
    h = jnp.maximum(h, 0.0)
    h = jnp.maximum(_dot32(h, W2_ref[...]) + b2_ref[...], 0.0)
    out_ref[...] = _sigmoid(_dot32(h, W3_ref[...]) + b3_ref[...])


def _rc_call(rows, p, rc_w, block_p):
    grid = (p // block_p,)
    voff = p // block_p
    in_specs = [
        pl.BlockSpec((block_p, _H), lambda i: (i, 0)),
        pl.BlockSpec((block_p, _H), lambda i, _v=voff: (i + _v, 0)),
    ] + [_full_spec(w.shape) for w in rc_w]
    return pl.pallas_call(
        _rc_body,
        grid=grid,
        in_specs=in_specs,
        out_specs=pl.BlockSpec((block_p, 1), lambda i: (i, 0)),
        out_shape=jax.ShapeDtypeStruct((p, 1), _F32),
        compiler_params=pltpu.CompilerParams(
            dimension_semantics=("arbitrary",)),
    )(rows, rows, *rc_w)


def _pick_block(n, target, align=8):
    b = min(target, n)
    b -= b % align
    while b > align and (n % b or b % align):
        b -= align
    return max(b, align)


def kernel(x, forward_level, gate, rc_pair_index, params):
    n = x.shape[0]
    p = rc_pair_index.shape[1]
    pipe_w, prob_w, rc_w = _fold_all(params)
    fl = forward_level.astype(jnp.int32)
    g = gate.astype(jnp.int32)
    act = (fl >= 1) & (fl <= _NUM_LEVELS - 1)
    sel = jnp.broadcast_to(jnp.where(act, g, 0).astype(jnp.int8)[:, None],
                           (n, _H))

    block_n = _pick_block(n, 4000, align=32)
    hs, hf, prob, hs2 = _main_call(x, sel, pipe_w, prob_w, block_n)

    info = plsc.get_sparse_core_info()
    align = 8 * info.num_cores * info.num_subcores
    idx = rc_pair_index.astype(jnp.int32).reshape(-1)
    pad = (-idx.shape[0]) % align
    if pad:
        idx = jnp.pad(idx, (0, pad))
    rows = _make_sc_gather(idx.shape[0], _H)(hs2, idx)

    block_p = _pick_block(p, 2000)
    is_rc = _rc_call(rows, p, rc_w, block_p)
    return (hs, hf, prob, is_rc)


# speedup vs baseline: 1.5599x; 1.1464x over previous
import functools

import jax
import jax.numpy as jnp
import numpy as np
from jax import lax
from jax.experimental import pallas as pl
from jax.experimental.pallas import tpu as pltpu
from jax.experimental.pallas import tpu_sc as plsc

_H = 128
_NUM_LEVELS = 8
_F32 = jnp.float32
_BF16 = jnp.bfloat16
_BN_INV = np.float32(1.0 / np.sqrt(1.0 + 1e-5))


def _dot32(a, w):
    return jnp.dot(a, w, preferred_element_type=_F32)


def _sigmoid(x):
    return 0.5 * (jnp.tanh(0.5 * x) + 1.0)


def _fold_mlp(p, bn=False, func=False):
    W1, b1 = p['W1'], p['b1']
    if func:
        b1 = b1 + W1[:, _H:].sum(axis=1)
        W1 = W1[:, :_H]
    W2, b2, W3, b3 = p['W2'], p['b2'], p['W3'], p['b3']
    if bn:
        s1 = p['g1'] * _BN_INV
        b1 = s1 * b1 + p['be1']
        W1 = W1 * s1[:, None]
        s2 = p['g2'] * _BN_INV
        b2 = s2 * b2 + p['be2']
        W2 = W2 * s2[:, None]
    return W1.T, b1, W2.T, b2, W3.T, b3


def _fold_pipe(p_mlp, p_gru, func=False):
    W1, b1, W2, b2, W3, b3 = _fold_mlp(p_mlp, func=func)
    Wih, bih = p_gru['W_ih'], p_gru['b_ih']
    ghc = p_gru['W_hh'].sum(axis=1) + p_gru['b_hh']
    beta = bih + Wih @ b3
    beta = beta.at[:2 * _H].add(ghc[:2 * _H])
    cn = ghc[2 * _H:]
    Wc = W3 @ Wih.T
    return W1, b1, W2, b2, Wc, beta, cn


def _fold_all(params):
    pipes = [
        _fold_pipe(params['aggr_and_strc'], params['update_and_strc']),
        _fold_pipe(params['aggr_not_strc'], params['update_not_strc']),
        _fold_pipe(params['aggr_and_func'], params['update_and_func'], func=True),
        _fold_pipe(params['aggr_not_func'], params['update_not_func'], func=True),
    ]
    W1cat = jnp.concatenate([p[0] for p in pipes], axis=1)
    b1 = jnp.stack([p[1] for p in pipes])[:, None, :]
    W2 = jnp.stack([p[2] for p in pipes])
    b2 = jnp.stack([p[3] for p in pipes])[:, None, :]
    Wc = jnp.stack([p[4] for p in pipes])
    beta = jnp.stack([p[5] for p in pipes])[:, None, :]
    cn = jnp.stack([p[6] for p in pipes])[:, None, :]

    Wp1, bp1, Wp2, bp2, Wp3, bp3 = _fold_mlp(params['readout_prob'], bn=True)
    prob_w = (Wp1, bp1[None, :], Wp2, bp2[None, :], Wp3, bp3[None, :])

    Wr1, br1, Wr2, br2, Wr3, br3 = _fold_mlp(params['readout_rc'], bn=True)
    rc_w = (Wr1[:_H], Wr1[_H:], br1[None, :], Wr2, br2[None, :], Wr3, br3[None, :])
    return (W1cat, b1, W2, b2, Wc, beta, cn), prob_w, rc_w


def _eye128():
    r = lax.broadcasted_iota(jnp.int32, (_H, _H), 0)
    c = lax.broadcasted_iota(jnp.int32, (_H, _H), 1)
    return (r == c).astype(_F32)


def _main_body(x_ref, sel_ref,
               W1_ref, b1_ref, W2_ref, b2_ref, Wc_ref, beta_ref, cn_ref,
               Wp1_ref, bp1_ref, Wp2_ref, bp2_ref, Wp3_ref, bp3_ref,
               hs_ref, hf_ref, prob_ref, hs2_ref):
    xb = x_ref[...]
    ident = _eye128()
    mb = sel_ref[0]
    mt = lax.dot_general(ident, mb, (((1,), (1,)), ((), ())),
                         preferred_element_type=_F32)
    nchunk = mb.shape[0]
    c = jnp.concatenate([mt[:, s:s + 1] for s in range(nchunk)], axis=0)
    m_and = c == 1.0
    m_not = c == 2.0

    h1all = _dot32(xb, W1_ref[...])
    outs = []
    for t in range(4):
        h = jnp.maximum(h1all[:, t * _H:(t + 1) * _H] + b1_ref[t], 0.0)
        h = jnp.maximum(_dot32(h, W2_ref[t]) + b2_ref[t], 0.0)
        gi = _dot32(h, Wc_ref[t]) + beta_ref[t]
        r = _sigmoid(gi[:, :_H])
        z = _sigmoid(gi[:, _H:2 * _H])
        n = jnp.tanh(gi[:, 2 * _H:] + r * cn_ref[t])
        outs.append((1.0 - z) * n + z)

    hs = jnp.where(m_and, outs[0], jnp.where(m_not, outs[1], 1.0))
    hf = jnp.where(m_and, outs[2], jnp.where(m_not, outs[3], 1.0))
    hs_ref[...] = hs
    hf_ref[...] = hf
    hs2_ref[...] = hs

    ph = jnp.maximum(_dot32(hf, Wp1_ref[...]) + bp1_ref[...], 0.0)
    ph = jnp.maximum(_dot32(ph, Wp2_ref[...]) + bp2_ref[...], 0.0)
    p_col = _dot32(ph, Wp3_ref[...]) + bp3_ref[...]
    chunks = [lax.dot_general(p_col[s * _H:(s + 1) * _H, :], ident,
                              (((0,), (0,)), ((), ())),
                              preferred_element_type=_F32)
              for s in range(nchunk)]
    prob_ref[...] = jnp.concatenate(chunks, axis=1)


def _full_spec(shape):
    nd = len(shape)
    return pl.BlockSpec(shape, lambda i, _nd=nd: (0,) * _nd)


def _main_call(x, sel3, pipe_w, prob_w, block_n):
    n = x.shape[0]
    nb = sel3.shape[0]
    grid = (nb,)
    weights = list(pipe_w) + list(prob_w)
    in_specs = [
        pl.BlockSpec((block_n, _H), lambda i: (i, 0)),
        pl.BlockSpec((1, block_n // _H, _H), lambda i: (i, 0, 0)),
    ] + [_full_spec(w.shape) for w in weights]
    out_specs = [
        pl.BlockSpec((block_n, _H), lambda i: (i, 0)),
        pl.BlockSpec((block_n, _H), lambda i: (i, 0)),
        pl.BlockSpec((1, block_n), lambda i: (0, i)),
        pl.BlockSpec((block_n, _H), lambda i: (i, 0)),
    ]
    out_shape = [
        jax.ShapeDtypeStruct((n, _H), _F32),
        jax.ShapeDtypeStruct((n, _H), _F32),
        jax.ShapeDtypeStruct((1, nb * block_n), _F32),
        jax.ShapeDtypeStruct((n, _H), _F32),
    ]
    return pl.pallas_call(
        _main_body,
        grid=grid,
        in_specs=in_specs,
        out_specs=out_specs,
        out_shape=out_shape,
        compiler_params=pltpu.CompilerParams(
            dimension_semantics=("arbitrary",)),
    )(x, sel3, *weights)


@functools.cache
def _make_sc_gather(num_rows, d):
    info = plsc.get_sparse_core_info()
    nw = info.num_cores * info.num_subcores
    b_per_w = num_rows // nw
    mesh = plsc.VectorSubcoreMesh(core_axis_name="c", subcore_axis_name="s")

    @functools.partial(
        pl.kernel,
        out_type=jax.ShapeDtypeStruct((num_rows, d), _F32),
        mesh=mesh,
        scratch_types=[
            pltpu.VMEM((b_per_w,), jnp.int32),
            pltpu.VMEM((b_per_w, d), _F32),
            pltpu.SemaphoreType.DMA,
        ],
    )
    def gather(table_hbm, idx_hbm, out_hbm, idx_v, rows_v, sem):
        wid = lax.axis_index("s") * info.num_cores + lax.axis_index("c")
        base = wid * b_per_w
        pltpu.sync_copy(idx_hbm.at[pl.ds(base, b_per_w)], idx_v)
        pltpu.async_copy(table_hbm.at[idx_v], rows_v, sem).wait()
        pltpu.sync_copy(rows_v, out_hbm.at[pl.ds(base, b_per_w)])

    return gather


def _rc_body(u_ref, v_ref, A1_ref, B1_ref, b1_ref, W2_ref, b2_ref,
             W3_ref, b3_ref, out_ref):
    h = _dot32(u_ref[...], A1_ref[...]) + _dot32(v_ref[...], B1_ref[...]) + b1_ref[...]
    h = jnp.maximum(h, 0.0)
    h = jnp.maximum(_dot32(h, W2_ref[...]) + b2_ref[...], 0.0)
    out_ref[...] = _sigmoid(_dot32(h, W3_ref[...]) + b3_ref[...])


def _rc_call(rows, p, rc_w, block_p):
    grid = (p // block_p,)
    voff = p // block_p
    in_specs = [
        pl.BlockSpec((block_p, _H), lambda i: (i, 0)),
        pl.BlockSpec((block_p, _H), lambda i, _v=voff: (i + _v, 0)),
    ] + [_full_spec(w.shape) for w in rc_w]
    return pl.pallas_call(
        _rc_body,
        grid=grid,
        in_specs=in_specs,
        out_specs=pl.BlockSpec((block_p, 1), lambda i: (i, 0)),
        out_shape=jax.ShapeDtypeStruct((p, 1), _F32),
        compiler_params=pltpu.CompilerParams(
            dimension_semantics=("arbitrary",)),
    )(rows, rows, *rc_w)


def _pick_block(n, target, align=8):
    b = min(target, n)
    b -= b % align
    while b > align and (n % b or b % align):
        b -= align
    return max(b, align)


def kernel(x, forward_level, gate, rc_pair_index, params):
    n = x.shape[0]
    p = rc_pair_index.shape[1]
    pipe_w, prob_w, rc_w = _fold_all(params)
    fl = forward_level.astype(jnp.int32)
    g = gate.astype(jnp.int32)
    act = (fl >= 1) & (fl <= _NUM_LEVELS - 1)
    sel_f = jnp.where(act, g, 0).astype(_F32)
    block_n = 32 * _H
    nb = -(-n // block_n)
    sel3 = jnp.pad(sel_f, (0, nb * block_n - n)).reshape(nb, block_n // _H, _H)

    hs, hf, prob_row, hs2 = _main_call(x, sel3, pipe_w, prob_w, block_n)
    prob = prob_row.reshape(-1)[:n].reshape(n, 1)

    info = plsc.get_sparse_core_info()
    align = 8 * info.num_cores * info.num_subcores
    idx = rc_pair_index.astype(jnp.int32).reshape(-1)
    pad = (-idx.shape[0]) % align
    if pad:
        idx = jnp.pad(idx, (0, pad))
    rows = _make_sc_gather(idx.shape[0], _H)(hs2, idx)

    block_p = _pick_block(p, 2000)
    is_rc = _rc_call(rows, p, rc_w, block_p)
    return (hs, hf, prob, is_rc)


# speedup vs baseline: 1.6226x; 1.0402x over previous
import functools

import jax
import jax.numpy as jnp
import numpy as np
from jax import lax
from jax.experimental import pallas as pl
from jax.experimental.pallas import tpu as pltpu
from jax.experimental.pallas import tpu_sc as plsc

_H = 128
_NUM_LEVELS = 8
_F32 = jnp.float32
_BF16 = jnp.bfloat16
_BN_INV = np.float32(1.0 / np.sqrt(1.0 + 1e-5))


def _dot32(a, w):
    return jnp.dot(a, w, preferred_element_type=_F32)


def _sigmoid(x):
    return 0.5 * (jnp.tanh(0.5 * x) + 1.0)


def _fold_mlp(p, bn=False):
    W1, b1 = p['W1'], p['b1']
    W2, b2, W3, b3 = p['W2'], p['b2'], p['W3'], p['b3']
    if bn:
        s1 = p['g1'] * _BN_INV
        b1 = s1 * b1 + p['be1']
        W1 = W1 * s1[:, None]
        s2 = p['g2'] * _BN_INV
        b2 = s2 * b2 + p['be2']
        W2 = W2 * s2[:, None]
    return W1.T, b1, W2.T, b2, W3.T, b3


def _fold_all(params):
    mlps = [params['aggr_and_strc'], params['aggr_not_strc'],
            params['aggr_and_func'], params['aggr_not_func']]
    grus = [params['update_and_strc'], params['update_not_strc'],
            params['update_and_func'], params['update_not_func']]
    W1s = jnp.stack([mlps[0]['W1'], mlps[1]['W1'],
                     mlps[2]['W1'][:, :_H], mlps[3]['W1'][:, :_H]])
    b1s = jnp.stack([mlps[0]['b1'], mlps[1]['b1'],
                     mlps[2]['b1'] + mlps[2]['W1'][:, _H:].sum(1),
                     mlps[3]['b1'] + mlps[3]['W1'][:, _H:].sum(1)])
    W2s = jnp.stack([m['W2'] for m in mlps])
    b2s = jnp.stack([m['b2'] for m in mlps])
    W3s = jnp.stack([m['W3'] for m in mlps])
    b3s = jnp.stack([m['b3'] for m in mlps])
    Wihs = jnp.stack([g['W_ih'] for g in grus])
    ghcs = (jnp.stack([g['W_hh'] for g in grus]).sum(-1)
            + jnp.stack([g['b_hh'] for g in grus]))
    betas = jnp.stack([g['b_ih'] for g in grus]) + jnp.einsum(
        'tgk,tk->tg', Wihs, b3s)
    betas = betas.at[:, :2 * _H].add(ghcs[:, :2 * _H])
    cn = ghcs[:, None, 2 * _H:]

    W1cat = W1s.transpose(2, 0, 1).reshape(_H, 4 * _H).astype(_BF16)
    W2 = jnp.swapaxes(W2s, 1, 2).astype(_BF16)
    Wc = jnp.swapaxes(jnp.einsum('tgo,tok->tgk', Wihs, W3s),
                      1, 2).astype(_BF16)
    b1 = b1s[:, None, :]
    b2 = b2s[:, None, :]
    beta = betas[:, None, :]

    Wp1, bp1, Wp2, bp2, Wp3, bp3 = _fold_mlp(params['readout_prob'], bn=True)
    prob_w = (Wp1, bp1[None, :], Wp2, bp2[None, :], Wp3, bp3[None, :])

    Wr1, br1, Wr2, br2, Wr3, br3 = _fold_mlp(params['readout_rc'], bn=True)
    rc_w = (Wr1[:_H], Wr1[_H:], br1[None, :], Wr2, br2[None, :], Wr3, br3[None, :])
    return (W1cat, b1, W2, b2, Wc, beta, cn), prob_w, rc_w


def _eye128():
    r = lax.broadcasted_iota(jnp.int32, (_H, _H), 0)
    c = lax.broadcasted_iota(jnp.int32, (_H, _H), 1)
    return (r == c).astype(_F32)


def _main_body(x_ref, sel_ref,
               W1_ref, b1_ref, W2_ref, b2_ref, Wc_ref, beta_ref, cn_ref,
               Wp1_ref, bp1_ref, Wp2_ref, bp2_ref, Wp3_ref, bp3_ref,
               hs_ref, hf_ref, prob_ref, hs2_ref):
    xb = x_ref[...]
    ident = _eye128()
    mb = sel_ref[0]
    mt = lax.dot_general(ident, mb, (((1,), (1,)), ((), ())),
                         preferred_element_type=_F32)
    nchunk = mb.shape[0]
    c = jnp.concatenate([mt[:, s:s + 1] for s in range(nchunk)], axis=0)
    m_and = c == 1.0
    m_not = c == 2.0

    h1all = jnp.dot(xb.astype(_BF16), W1_ref[...], preferred_element_type=_F32)
    outs = []
    for t in range(4):
        h = jnp.maximum(h1all[:, t * _H:(t + 1) * _H] + b1_ref[t], 0.0)
        h = jnp.maximum(jnp.dot(h.astype(_BF16), W2_ref[t], preferred_element_type=_F32) + b2_ref[t], 0.0)
        gi = jnp.dot(h.astype(_BF16), Wc_ref[t], preferred_element_type=_F32) + beta_ref[t]
        r = _sigmoid(gi[:, :_H])
        z = _sigmoid(gi[:, _H:2 * _H])
        n = jnp.tanh(gi[:, 2 * _H:] + r * cn_ref[t])
        outs.append((1.0 - z) * n + z)

    hs = jnp.where(m_and, outs[0], jnp.where(m_not, outs[1], 1.0))
    hf = jnp.where(m_and, outs[2], jnp.where(m_not, outs[3], 1.0))
    hs_ref[...] = hs
    hf_ref[...] = hf
    hs2_ref[...] = hs

    ph = jnp.maximum(_dot32(hf, Wp1_ref[...]) + bp1_ref[...], 0.0)
    ph = jnp.maximum(_dot32(ph, Wp2_ref[...]) + bp2_ref[...], 0.0)
    p_col = _dot32(ph, Wp3_ref[...]) + bp3_ref[...]
    chunks = [lax.dot_general(p_col[s * _H:(s + 1) * _H, :], ident,
                              (((0,), (0,)), ((), ())),
                              preferred_element_type=_F32)
              for s in range(nchunk)]
    prob_ref[...] = jnp.concatenate(chunks, axis=1)


def _full_spec(shape):
    nd = len(shape)
    return pl.BlockSpec(shape, lambda i, _nd=nd: (0,) * _nd)


def _main_call(x, sel3, pipe_w, prob_w, block_n):
    n = x.shape[0]
    nb = sel3.shape[0]
    grid = (nb,)
    weights = list(pipe_w) + list(prob_w)
    in_specs = [
        pl.BlockSpec((block_n, _H), lambda i: (i, 0)),
        pl.BlockSpec((1, block_n // _H, _H), lambda i: (i, 0, 0)),
    ] + [_full_spec(w.shape) for w in weights]
    out_specs = [
        pl.BlockSpec((block_n, _H), lambda i: (i, 0)),
        pl.BlockSpec((block_n, _H), lambda i: (i, 0)),
        pl.BlockSpec((1, block_n), lambda i: (0, i)),
        pl.BlockSpec((block_n, _H), lambda i: (i, 0)),
    ]
    out_shape = [
        jax.ShapeDtypeStruct((n, _H), _F32),
        jax.ShapeDtypeStruct((n, _H), _F32),
        jax.ShapeDtypeStruct((1, nb * block_n), _F32),
        jax.ShapeDtypeStruct((n, _H), _F32),
    ]
    return pl.pallas_call(
        _main_body,
        grid=grid,
        in_specs=in_specs,
        out_specs=out_specs,
        out_shape=out_shape,
        compiler_params=pltpu.CompilerParams(
            dimension_semantics=("arbitrary",)),
    )(x, sel3, *weights)


@functools.cache
def _make_sc_gather(num_rows, d):
    info = plsc.get_sparse_core_info()
    nw = info.num_cores * info.num_subcores
    b_per_w = num_rows // nw
    mesh = plsc.VectorSubcoreMesh(core_axis_name="c", subcore_axis_name="s")

    @functools.partial(
        pl.kernel,
        out_type=jax.ShapeDtypeStruct((num_rows, d), _F32),
        mesh=mesh,
        scratch_types=[
            pltpu.VMEM((b_per_w,), jnp.int32),
            pltpu.VMEM((b_per_w, d), _F32),
            pltpu.SemaphoreType.DMA,
        ],
    )
    def gather(table_hbm, idx_hbm, out_hbm, idx_v, rows_v, sem):
        wid = lax.axis_index("s") * info.num_cores + lax.axis_index("c")
        base = wid * b_per_w
        pltpu.sync_copy(idx_hbm.at[pl.ds(base, b_per_w)], idx_v)
        pltpu.async_copy(table_hbm.at[idx_v], rows_v, sem).wait()
        pltpu.sync_copy(rows_v, out_hbm.at[pl.ds(base, b_per_w)])

    return gather


def _rc_body(u_ref, v_ref, A1_ref, B1_ref, b1_ref, W2_ref, b2_ref,
             W3_ref, b3_ref, out_ref):
    h = _dot32(u_ref[...], A1_ref[...]) + _dot32(v_ref[...], B1_ref[...]) + b1_ref[...]
    h = jnp.maximum(h, 0.0)
    h = jnp.maximum(_dot32(h, W2_ref[...]) + b2_ref[...], 0.0)
    out_ref[...] = _sigmoid(_dot32(h, W3_ref[...]) + b3_ref[...])


def _rc_call(rows, p, rc_w, block_p):
    grid = (p // block_p,)
    voff = p // block_p
    in_specs = [
        pl.BlockSpec((block_p, _H), lambda i: (i, 0)),
        pl.BlockSpec((block_p, _H), lambda i, _v=voff: (i + _v, 0)),
    ] + [_full_spec(w.shape) for w in rc_w]
    return pl.pallas_call(
        _rc_body,
        grid=grid,
        in_specs=in_specs,
        out_specs=pl.BlockSpec((block_p, 1), lambda i: (i, 0)),
        out_shape=jax.ShapeDtypeStruct((p, 1), _F32),
        compiler_params=pltpu.CompilerParams(
            dimension_semantics=("arbitrary",)),
    )(rows, rows, *rc_w)


def _pick_block(n, target, align=8):
    b = min(target, n)
    b -= b % align
    while b > align and (n % b or b % align):
        b -= align
    return max(b, align)


def kernel(x, forward_level, gate, rc_pair_index, params):
    n = x.shape[0]
    p = rc_pair_index.shape[1]
    pipe_w, prob_w, rc_w = _fold_all(params)
    fl = forward_level.astype(jnp.int32)
    g = gate.astype(jnp.int32)
    act = (fl >= 1) & (fl <= _NUM_LEVELS - 1)
    sel_f = jnp.where(act, g, 0).astype(_F32)
    block_n = 32 * _H
    nb = -(-n // block_n)
    sel3 = jnp.pad(sel_f, (0, nb * block_n - n)).reshape(nb, block_n // _H, _H)

    hs, hf, prob_row, hs2 = _main_call(x, sel3, pipe_w, prob_w, block_n)
    prob = prob_row.reshape(-1)[:n].reshape(n, 1)

    info = plsc.get_sparse_core_info()
    align = 8 * info.num_cores * info.num_subcores
    idx = rc_pair_index.astype(jnp.int32).reshape(-1)
    pad = (-idx.shape[0]) % align
    if pad:
        idx = jnp.pad(idx, (0, pad))
    rows = _make_sc_gather(idx.shape[0], _H)(hs2, idx)

    block_p = _pick_block(p, 2000)
    is_rc = _rc_call(rows, p, rc_w, block_p)
    return (hs, hf, prob, is_rc)
